# baseline (speedup 1.0000x reference)
import functools

import jax
import jax.numpy as jnp
from jax import lax
from jax.experimental import pallas as pl
from jax.experimental.pallas import tpu as pltpu
from jax.experimental.pallas import tpu_sc as plsc

B = 16384
T = 20
EMBED_DIM = 16

NC = 2
NS = 16
NW = NC * NS

NTOT = B * T
PER_W = NTOT // NW
CH = 128
NCHUNK = PER_W // CH
K = 8


def _make_kernel():
    mesh = plsc.VectorSubcoreMesh(core_axis_name="c", subcore_axis_name="s")

    @functools.partial(
        pl.kernel,
        mesh=mesh,
        compiler_params=pltpu.CompilerParams(use_tc_tiling_on_sc=False),
        out_type=jax.ShapeDtypeStruct((NTOT, EMBED_DIM), jnp.float32),
        scratch_types=[
            pltpu.VMEM((NCHUNK, CH), jnp.int32),
            pltpu.VMEM((K, CH, EMBED_DIM), jnp.float32),
            pltpu.SemaphoreType.DMA,
        ],
    )
    def body(idx_hbm, wts_hbm, out_hbm, idx_v, rows_v, sem):
        wid = lax.axis_index("s") * NC + lax.axis_index("c")
        base = wid * PER_W
        pltpu.sync_copy(idx_hbm.at[wid], idx_v)

        def group(g, carry):
            j0 = g * K
            copies = []
            for b in range(K):
                copies.append(
                    pltpu.async_copy(
                        wts_hbm.at[idx_v.at[j0 + b]], rows_v.at[b], sem
                    )
                )
            for b in range(K):
                copies[b].wait()
            for b in range(K):
                pltpu.sync_copy(
                    rows_v.at[b],
                    out_hbm.at[pl.ds(base + (j0 + b) * CH, CH)],
                )
            return carry

        lax.fori_loop(0, NCHUNK // K, group, 0)

    return body


_gather_kernel = _make_kernel()


@jax.jit
def kernel(x, wts):
    idx = x.reshape(NW, NCHUNK, CH).astype(jnp.int32)
    out = _gather_kernel(idx, wts)
    return out.reshape(B, T, EMBED_DIM)


# speedup vs baseline: 1.0092x; 1.0092x over previous
import functools

import jax
import jax.numpy as jnp
from jax import lax
from jax.experimental import pallas as pl
from jax.experimental.pallas import tpu as pltpu
from jax.experimental.pallas import tpu_sc as plsc

B = 16384
T = 20
EMBED_DIM = 16

NC = 2
NS = 16
NW = NC * NS

NTOT = B * T
PER_W = NTOT // NW
CH = 512
NCHUNK = PER_W // CH
K = 4


def _make_kernel():
    mesh = plsc.VectorSubcoreMesh(core_axis_name="c", subcore_axis_name="s")

    @functools.partial(
        pl.kernel,
        mesh=mesh,
        compiler_params=pltpu.CompilerParams(use_tc_tiling_on_sc=False),
        out_type=jax.ShapeDtypeStruct((NTOT, EMBED_DIM), jnp.float32),
        scratch_types=[
            pltpu.VMEM((NCHUNK, CH), jnp.int32),
            pltpu.VMEM((K, CH, EMBED_DIM), jnp.float32),
            pltpu.SemaphoreType.DMA,
        ],
    )
    def body(idx_hbm, wts_hbm, out_hbm, idx_v, rows_v, sem):
        wid = lax.axis_index("s") * NC + lax.axis_index("c")
        base = wid * PER_W
        pltpu.sync_copy(idx_hbm.at[wid], idx_v)

        def group(g, carry):
            j0 = g * K
            copies = []
            for b in range(K):
                copies.append(
                    pltpu.async_copy(
                        wts_hbm.at[idx_v.at[j0 + b]], rows_v.at[b], sem
                    )
                )
            for b in range(K):
                copies[b].wait()
            for b in range(K):
                pltpu.sync_copy(
                    rows_v.at[b],
                    out_hbm.at[pl.ds(base + (j0 + b) * CH, CH)],
                )
            return carry

        lax.fori_loop(0, NCHUNK // K, group, 0)

    return body


_gather_kernel = _make_kernel()


@jax.jit
def kernel(x, wts):
    idx = x.reshape(NW, NCHUNK, CH).astype(jnp.int32)
    out = _gather_kernel(idx, wts)
    return out.reshape(B, T, EMBED_DIM)


# speedup vs baseline: 1.4259x; 1.4129x over previous
import functools

import jax
import jax.numpy as jnp
from jax import lax
from jax.experimental import pallas as pl
from jax.experimental.pallas import tpu as pltpu
from jax.experimental.pallas import tpu_sc as plsc

B = 16384
T = 20
EMBED_DIM = 16
INPUT_DIM = 1000000

NC = 2
NS = 16
NW = NC * NS

CH = 128
NBB = B // CH // NW
NCH = T * NBB
K = 4
NG = NCH // K
HB = EMBED_DIM // 8


def _make_kernel():
    mesh = plsc.VectorSubcoreMesh(core_axis_name="c", subcore_axis_name="s")

    @functools.partial(
        pl.kernel,
        mesh=mesh,
        compiler_params=pltpu.CompilerParams(
            use_tc_tiling_on_sc=False, needs_layout_passes=False
        ),
        out_type=jax.ShapeDtypeStruct((T, HB, B // CH, 8, CH), jnp.float32),
        scratch_types=[
            pltpu.VMEM((T, NBB, CH), jnp.int32),
            pltpu.VMEM((2, K, CH, EMBED_DIM), jnp.float32),
            pltpu.VMEM((2, K, HB, 8, CH), jnp.float32),
            pltpu.SemaphoreType.DMA,
            pltpu.SemaphoreType.DMA,
        ],
    )
    def body(idx_hbm, wts_hbm, out_hbm, idx_v, rows_v, tbuf_v, gsem, wsem):
        wid = lax.axis_index("s") * NC + lax.axis_index("c")
        bbb = wid * NBB
        pltpu.sync_copy(idx_hbm.at[wid], idx_v)

        lane = lax.iota(jnp.int32, 16)

        def fire(j, p, b):
            t, cb = lax.div(j, NBB), lax.rem(j, NBB)
            pltpu.async_copy(
                wts_hbm.at[idx_v.at[t, cb]], rows_v.at[p, b], gsem
            )

        for b in range(K):
            fire(b, 0, b)

        def group(g, carry):
            p = lax.rem(g, 2)
            pn = 1 - p

            @pl.when(g + 1 < NG)
            def _():
                for b in range(K):
                    fire((g + 1) * K + b, pn, b)

            for b in range(K):
                pltpu.make_async_copy(
                    out_hbm.at[0, 0, 0], rows_v.at[p, b], gsem
                ).wait()

            @pl.when(g >= 2)
            def _():
                for b in range(K):
                    for hb in range(HB):
                        pltpu.make_async_copy(
                            out_hbm.at[0, 0, 0], tbuf_v.at[p, b, hb], wsem
                        ).wait()

            for b in range(K):
                j = g * K + b
                t, cb = lax.div(j, NBB), lax.rem(j, NBB)
                blk = rows_v.at[p, b]
                for hb in range(HB):
                    for r in range(8):
                        h = jnp.full((16,), hb * 8 + r, jnp.int32)
                        for c0 in range(CH // 16):
                            vec = plsc.load_gather(blk, [c0 * 16 + lane, h])
                            tbuf_v[p, b, hb, r, pl.ds(c0 * 16, 16)] = vec
                for hb in range(HB):
                    pltpu.async_copy(
                        tbuf_v.at[p, b, hb], out_hbm.at[t, hb, bbb + cb], wsem
                    )
            return carry

        lax.fori_loop(0, NG, group, 0)

        for _ in range(min(2, NG) * K * HB):
            pltpu.make_async_copy(
                out_hbm.at[0, 0, 0], tbuf_v.at[0, 0, 0], wsem
            ).wait()

    return body


_gather_kernel = _make_kernel()

_TBLK = 8192


def _tp_body(x_ref, o_ref):
    o_ref[...] = x_ref[...].T


_tc_transpose = pl.pallas_call(
    _tp_body,
    grid=(pl.cdiv(INPUT_DIM, _TBLK),),
    in_specs=[pl.BlockSpec((EMBED_DIM, _TBLK), lambda i: (0, i))],
    out_specs=pl.BlockSpec((_TBLK, EMBED_DIM), lambda i: (i, 0)),
    out_shape=jax.ShapeDtypeStruct((INPUT_DIM, EMBED_DIM), jnp.float32),
)


@jax.jit
def kernel(x, wts):
    idx = (
        x.astype(jnp.int32)
        .reshape(NW, NBB * CH, T)
        .transpose(0, 2, 1)
        .reshape(NW, T, NBB, CH)
    )
    out5 = _gather_kernel(idx, _tc_transpose(wts.T))
    return out5.transpose(2, 4, 0, 1, 3).reshape(B, T, EMBED_DIM)
